# baseline (speedup 1.0000x reference)
import functools

import jax
import jax.numpy as jnp
from jax import lax
from jax.experimental import pallas as pl
from jax.experimental.pallas import tpu as pltpu
from jax.experimental.pallas import tpu_sc as plsc

_N = 10000
_E = 320000
_D = 128
_NC = 2
_NS = 16
_NW = _NC * _NS
_K = 128
_EW = 80
_CH = 16
_EPT = _EW * _K
_EP = _NW * _EPT
_NP = 10240
_STRIPE = _NP // _NS


@functools.cache
def _sc_mesh():
    return plsc.VectorSubcoreMesh(core_axis_name="c", subcore_axis_name="s",
                                  num_cores=_NC, num_subcores=_NS)


@functools.cache
def _sc_degree_kernel():
    @functools.partial(
        pl.kernel,
        out_type=jax.ShapeDtypeStruct((_NC, _NP), jnp.float32),
        mesh=_sc_mesh(),
        scratch_types=[
            pltpu.VMEM((_EW, _K), jnp.int32),
            pltpu.VMEM((_K,), jnp.float32),
            pltpu.VMEM_SHARED((_NP,), jnp.float32),
        ],
    )
    def deg_kernel(edge_hbm, tail_hbm, ones_hbm, zeros_hbm, out_hbm,
                   dst_v, ones_v, deg_sh):
        c = lax.axis_index("c")
        s = lax.axis_index("s")
        wid = c * _NS + s
        pltpu.sync_copy(zeros_hbm.at[pl.ds(s * _STRIPE, _STRIPE)],
                        deg_sh.at[pl.ds(s * _STRIPE, _STRIPE)])
        pltpu.sync_copy(ones_hbm, ones_v)

        @pl.when(wid < _NW - 1)
        def _():
            pltpu.sync_copy(edge_hbm.at[1].at[pl.ds(wid * _EW, _EW)], dst_v)

        @pl.when(wid == _NW - 1)
        def _():
            pltpu.sync_copy(edge_hbm.at[1].at[pl.ds(_MAIN - _CH, _CH)],
                            dst_v.at[pl.ds(0, _CH)])
            pltpu.sync_copy(tail_hbm.at[1], dst_v.at[pl.ds(_CH, _TW)])

        plsc.subcore_barrier()

        @pl.loop(0, _EW)
        def _(j):
            pltpu.sync_copy(ones_v, deg_sh.at[dst_v.at[j]], add=True)

        plsc.subcore_barrier()
        pltpu.sync_copy(deg_sh.at[pl.ds(s * _STRIPE, _STRIPE)],
                        out_hbm.at[c].at[pl.ds(s * _STRIPE, _STRIPE)])

    return deg_kernel


_MAIN = 2496
_TW = 64


@functools.cache
def _sc_aggregate_kernel():
    @functools.partial(
        pl.kernel,
        out_type=jax.ShapeDtypeStruct((_NC, _NP, _D), jnp.float32),
        mesh=_sc_mesh(),
        scratch_types=[
            pltpu.VMEM((_CH, _K), jnp.int32),
            pltpu.VMEM((_CH, _K), jnp.int32),
            pltpu.VMEM((_K, _D), jnp.float32),
            pltpu.VMEM((_K, _D), jnp.float32),
            pltpu.VMEM_SHARED((_NP, _D), jnp.float32),
            pltpu.SemaphoreType.DMA,
            pltpu.SemaphoreType.DMA,
            pltpu.SemaphoreType.DMA,
            pltpu.SemaphoreType.DMA,
        ],
    )
    def agg_kernel(g_hbm, edge_hbm, tail_hbm, zrow_hbm, out_hbm,
                   src_v, dst_v, rows_a, rows_b, acc_sh,
                   sem_a, sem_b, sem_sa, sem_sb):
        c = lax.axis_index("c")
        s = lax.axis_index("s")
        base = (c * _NS + s) * _EW
        pltpu.sync_copy(zrow_hbm, rows_a)

        @pl.loop(0, _STRIPE // _K)
        def _(i):
            pltpu.sync_copy(rows_a, acc_sh.at[pl.ds(s * _STRIPE + i * _K, _K)])

        plsc.subcore_barrier()

        @pl.loop(0, _EW // _CH)
        def _(ci):
            wstart = base + ci * _CH

            @pl.when(wstart < _MAIN)
            def _():
                pltpu.sync_copy(edge_hbm.at[0].at[pl.ds(wstart, _CH)], src_v)
                pltpu.sync_copy(edge_hbm.at[1].at[pl.ds(wstart, _CH)], dst_v)

            @pl.when(wstart >= _MAIN)
            def _():
                pltpu.sync_copy(
                    tail_hbm.at[0].at[pl.ds(wstart - _MAIN, _CH)], src_v)
                pltpu.sync_copy(
                    tail_hbm.at[1].at[pl.ds(wstart - _MAIN, _CH)], dst_v)

            pltpu.async_copy(g_hbm.at[src_v.at[0]], rows_a, sem_a)
            pltpu.async_copy(g_hbm.at[src_v.at[1]], rows_b, sem_b)

            @pl.loop(0, _CH, step=2)
            def _(j):
                pltpu.make_async_copy(
                    g_hbm.at[src_v.at[j]], rows_a, sem_a).wait()
                pltpu.async_copy(rows_a, acc_sh.at[dst_v.at[j]], sem_sa,
                                 add=True)
                pltpu.make_async_copy(
                    g_hbm.at[src_v.at[j + 1]], rows_b, sem_b).wait()
                pltpu.async_copy(rows_b, acc_sh.at[dst_v.at[j + 1]], sem_sb,
                                 add=True)
                pltpu.make_async_copy(
                    rows_a, acc_sh.at[dst_v.at[j]], sem_sa).wait()

                @pl.when(j + 2 < _CH)
                def _():
                    pltpu.async_copy(
                        g_hbm.at[src_v.at[j + 2]], rows_a, sem_a)

                pltpu.make_async_copy(
                    rows_b, acc_sh.at[dst_v.at[j + 1]], sem_sb).wait()

                @pl.when(j + 3 < _CH)
                def _():
                    pltpu.async_copy(
                        g_hbm.at[src_v.at[j + 3]], rows_b, sem_b)

        plsc.subcore_barrier()
        pltpu.sync_copy(acc_sh.at[pl.ds(s * _STRIPE, _STRIPE)],
                        out_hbm.at[c].at[pl.ds(s * _STRIPE, _STRIPE)])

    return agg_kernel


def _sc_degree(edge3, tail3, ones_k, zeros_n):
    return _sc_degree_kernel()(edge3, tail3, ones_k, zeros_n)


def _sc_aggregate(g, edge3, tail3, zeros_row):
    return _sc_aggregate_kernel()(g, edge3, tail3, zeros_row)


_B = 2000


def _dinv_body(deg_ref, o_ref):
    ones2 = jnp.ones((2, 1), jnp.float32)
    degsum = lax.dot_general(deg_ref[...], ones2, (((0,), (0,)), ((), ())),
                             preferred_element_type=jnp.float32)
    o_ref[...] = lax.rsqrt(degsum + 1.0)


def _tc_dinv(deg2):
    return pl.pallas_call(
        _dinv_body,
        grid=(1,),
        in_specs=[pl.BlockSpec((2, _NP), lambda i: (0, 0))],
        out_specs=pl.BlockSpec((_NP, 1), lambda i: (0, 0)),
        out_shape=jax.ShapeDtypeStruct((_NP, 1), jnp.float32),
    )(deg2)


def _mm_scale_body(dinv_ref, x_ref, w_ref, o_ref):
    dinv = dinv_ref[...]
    o_ref[...] = jnp.dot(x_ref[...], w_ref[...],
                         preferred_element_type=jnp.float32) * dinv


def _combine_mm_body(dinv_ref, p_ref, g_ref, b_ref, w_ref, o_ref):
    dinv = dinv_ref[...]
    h = jnp.maximum((p_ref[0] + p_ref[1] + g_ref[...]) * dinv + b_ref[...], 0.0)
    o_ref[...] = jnp.dot(h, w_ref[...],
                         preferred_element_type=jnp.float32) * dinv


def _combine_out_body(dinv_ref, p_ref, g_ref, b_ref, o_ref):
    dinv = dinv_ref[...]
    o_ref[...] = jnp.maximum(
        (p_ref[0] + p_ref[1] + g_ref[...]) * dinv + b_ref[...], 0.0)


def _tc_mm_scale(dinvc, x, w):
    return pl.pallas_call(
        _mm_scale_body,
        grid=(_N // _B,),
        in_specs=[
            pl.BlockSpec((_B, 1), lambda i: (i, 0)),
            pl.BlockSpec((_B, _D), lambda i: (i, 0)),
            pl.BlockSpec((_D, _D), lambda i: (0, 0)),
        ],
        out_specs=pl.BlockSpec((_B, _D), lambda i: (i, 0)),
        out_shape=jax.ShapeDtypeStruct((_N, _D), jnp.float32),
    )(dinvc, x, w)


def _tc_combine_mm(dinvc, p, g, b, w):
    return pl.pallas_call(
        _combine_mm_body,
        grid=(_N // _B,),
        in_specs=[
            pl.BlockSpec((_B, 1), lambda i: (i, 0)),
            pl.BlockSpec((_NC, _B, _D), lambda i: (0, i, 0)),
            pl.BlockSpec((_B, _D), lambda i: (i, 0)),
            pl.BlockSpec((1, _D), lambda i: (0, 0)),
            pl.BlockSpec((_D, _D), lambda i: (0, 0)),
        ],
        out_specs=pl.BlockSpec((_B, _D), lambda i: (i, 0)),
        out_shape=jax.ShapeDtypeStruct((_N, _D), jnp.float32),
    )(dinvc, p, g, b, w)


def _tc_combine_out(dinvc, p, g, b):
    return pl.pallas_call(
        _combine_out_body,
        grid=(_N // _B,),
        in_specs=[
            pl.BlockSpec((_B, 1), lambda i: (i, 0)),
            pl.BlockSpec((_NC, _B, _D), lambda i: (0, i, 0)),
            pl.BlockSpec((_B, _D), lambda i: (i, 0)),
            pl.BlockSpec((1, _D), lambda i: (0, 0)),
        ],
        out_specs=pl.BlockSpec((_B, _D), lambda i: (i, 0)),
        out_shape=jax.ShapeDtypeStruct((_N, _D), jnp.float32),
    )(dinvc, p, g, b)


def kernel(x, edge_index, W1, b1, W2, b2):
    ei = edge_index.astype(jnp.int32)
    edge3 = ei.reshape(2, _E // _K, _K)
    npad = _TW * _K - (_E - _MAIN * _K)
    it = jnp.arange(npad, dtype=jnp.int32)
    pad2 = jnp.stack([it % _N, _N + it % (_NP - _N)])
    tail3 = jnp.concatenate(
        [ei[:, _MAIN * _K:], pad2], axis=1).reshape(2, _TW, _K)

    ones_k = jnp.ones((_K,), jnp.float32)
    zeros_n = jnp.zeros((_NP,), jnp.float32)
    zeros_row = jnp.zeros((_K, _D), jnp.float32)

    deg2 = _sc_degree(edge3, tail3, ones_k, zeros_n)
    dinvc = _tc_dinv(deg2)

    b1r = b1.reshape(1, _D)
    b2r = b2.reshape(1, _D)

    g1 = _tc_mm_scale(dinvc, x, W1)
    p1 = _sc_aggregate(g1, edge3, tail3, zeros_row)
    g2 = _tc_combine_mm(dinvc, p1, g1, b1r, W2)
    p2 = _sc_aggregate(g2, edge3, tail3, zeros_row)
    return _tc_combine_out(dinvc, p2, g2, b2r)


# speedup vs baseline: 1.2889x; 1.2889x over previous
import functools

import jax
import jax.numpy as jnp
from jax import lax
from jax.experimental import pallas as pl
from jax.experimental.pallas import tpu as pltpu
from jax.experimental.pallas import tpu_sc as plsc

_N = 10000
_E = 320000
_D = 128
_NC = 2
_NS = 16
_NW = _NC * _NS
_K = 128
_EW = 80
_CH = 40
_EPT = _EW * _K
_EP = _NW * _EPT
_NP = 10240
_STRIPE = _NP // _NS


@functools.cache
def _sc_mesh():
    return plsc.VectorSubcoreMesh(core_axis_name="c", subcore_axis_name="s",
                                  num_cores=_NC, num_subcores=_NS)


@functools.cache
def _sc_degree_kernel():
    @functools.partial(
        pl.kernel,
        out_type=jax.ShapeDtypeStruct((_NC, _NP), jnp.float32),
        mesh=_sc_mesh(),
        scratch_types=[
            pltpu.VMEM((_EW, _K), jnp.int32),
            pltpu.VMEM((_K,), jnp.float32),
            pltpu.VMEM_SHARED((_NP,), jnp.float32),
        ],
    )
    def deg_kernel(edge_hbm, tail_hbm, ones_hbm, zeros_hbm, out_hbm,
                   dst_v, ones_v, deg_sh):
        c = lax.axis_index("c")
        s = lax.axis_index("s")
        wid = c * _NS + s
        pltpu.sync_copy(zeros_hbm.at[pl.ds(s * _STRIPE, _STRIPE)],
                        deg_sh.at[pl.ds(s * _STRIPE, _STRIPE)])
        pltpu.sync_copy(ones_hbm, ones_v)

        @pl.when(wid < _NW - 1)
        def _():
            pltpu.sync_copy(edge_hbm.at[1].at[pl.ds(wid * _EW, _EW)], dst_v)

        @pl.when(wid == _NW - 1)
        def _():
            pltpu.sync_copy(tail_hbm.at[1], dst_v)

        plsc.subcore_barrier()

        @pl.loop(0, _EW)
        def _(j):
            pltpu.sync_copy(ones_v, deg_sh.at[dst_v.at[j]], add=True)

        plsc.subcore_barrier()
        pltpu.sync_copy(deg_sh.at[pl.ds(s * _STRIPE, _STRIPE)],
                        out_hbm.at[c].at[pl.ds(s * _STRIPE, _STRIPE)])

    return deg_kernel


_MAIN = 2480
_TW = 80


@functools.cache
def _sc_aggregate_kernel():
    @functools.partial(
        pl.kernel,
        out_type=jax.ShapeDtypeStruct((_NC, _NP, _D), jnp.float32),
        mesh=_sc_mesh(),
        scratch_types=[
            pltpu.VMEM((_CH, _K), jnp.int32),
            pltpu.VMEM((_CH, _K), jnp.int32),
            pltpu.VMEM((_K, _D), jnp.float32),
            pltpu.VMEM((_K, _D), jnp.float32),
            pltpu.VMEM_SHARED((_NP, _D), jnp.float32),
            pltpu.SemaphoreType.DMA,
            pltpu.SemaphoreType.DMA,
        ],
    )
    def agg_kernel(g_hbm, edge_hbm, tail_hbm, zrow_hbm, out_hbm,
                   src_v, dst_v, rows_a, rows_b, acc_sh, sem_a, sem_b):
        c = lax.axis_index("c")
        s = lax.axis_index("s")
        base = (c * _NS + s) * _EW
        pltpu.sync_copy(zrow_hbm, rows_a)

        @pl.loop(0, _STRIPE // _K)
        def _(i):
            pltpu.sync_copy(rows_a, acc_sh.at[pl.ds(s * _STRIPE + i * _K, _K)])

        plsc.subcore_barrier()

        @pl.loop(0, _EW // _CH)
        def _(ci):
            wstart = base + ci * _CH

            @pl.when(wstart < _MAIN)
            def _():
                pltpu.sync_copy(edge_hbm.at[0].at[pl.ds(wstart, _CH)], src_v)
                pltpu.sync_copy(edge_hbm.at[1].at[pl.ds(wstart, _CH)], dst_v)

            @pl.when(wstart >= _MAIN)
            def _():
                pltpu.sync_copy(
                    tail_hbm.at[0].at[pl.ds(wstart - _MAIN, _CH)], src_v)
                pltpu.sync_copy(
                    tail_hbm.at[1].at[pl.ds(wstart - _MAIN, _CH)], dst_v)

            pltpu.async_copy(g_hbm.at[src_v.at[0]], rows_a, sem_a)
            pltpu.async_copy(g_hbm.at[src_v.at[1]], rows_b, sem_b)

            @pl.loop(0, _CH, step=2)
            def _(j):
                pltpu.make_async_copy(
                    g_hbm.at[src_v.at[j]], rows_a, sem_a).wait()
                pltpu.sync_copy(rows_a, acc_sh.at[dst_v.at[j]], add=True)

                @pl.when(j + 2 < _CH)
                def _():
                    pltpu.async_copy(
                        g_hbm.at[src_v.at[j + 2]], rows_a, sem_a)

                pltpu.make_async_copy(
                    g_hbm.at[src_v.at[j + 1]], rows_b, sem_b).wait()
                pltpu.sync_copy(rows_b, acc_sh.at[dst_v.at[j + 1]],
                                add=True)

                @pl.when(j + 3 < _CH)
                def _():
                    pltpu.async_copy(
                        g_hbm.at[src_v.at[j + 3]], rows_b, sem_b)

        plsc.subcore_barrier()
        pltpu.sync_copy(acc_sh.at[pl.ds(s * _STRIPE, _STRIPE)],
                        out_hbm.at[c].at[pl.ds(s * _STRIPE, _STRIPE)])

    return agg_kernel


def _sc_degree(edge3, tail3, ones_k, zeros_n):
    return _sc_degree_kernel()(edge3, tail3, ones_k, zeros_n)


def _sc_aggregate(g, edge3, tail3, zeros_row):
    return _sc_aggregate_kernel()(g, edge3, tail3, zeros_row)


_B = 2000


def _dinv_body(deg_ref, o_ref):
    ones2 = jnp.ones((2, 1), jnp.float32)
    degsum = lax.dot_general(deg_ref[...], ones2, (((0,), (0,)), ((), ())),
                             preferred_element_type=jnp.float32)
    o_ref[...] = lax.rsqrt(degsum + 1.0)


def _tc_dinv(deg2):
    return pl.pallas_call(
        _dinv_body,
        grid=(1,),
        in_specs=[pl.BlockSpec((2, _NP), lambda i: (0, 0))],
        out_specs=pl.BlockSpec((_NP, 1), lambda i: (0, 0)),
        out_shape=jax.ShapeDtypeStruct((_NP, 1), jnp.float32),
    )(deg2)


def _mm_scale_body(dinv_ref, x_ref, w_ref, o_ref):
    dinv = dinv_ref[...]
    o_ref[...] = jnp.dot(x_ref[...], w_ref[...],
                         preferred_element_type=jnp.float32) * dinv


def _combine_mm_body(dinv_ref, p_ref, g_ref, b_ref, w_ref, o_ref):
    dinv = dinv_ref[...]
    h = jnp.maximum((p_ref[0] + p_ref[1] + g_ref[...]) * dinv + b_ref[...], 0.0)
    o_ref[...] = jnp.dot(h, w_ref[...],
                         preferred_element_type=jnp.float32) * dinv


def _combine_out_body(dinv_ref, p_ref, g_ref, b_ref, o_ref):
    dinv = dinv_ref[...]
    o_ref[...] = jnp.maximum(
        (p_ref[0] + p_ref[1] + g_ref[...]) * dinv + b_ref[...], 0.0)


def _tc_mm_scale(dinvc, x, w):
    return pl.pallas_call(
        _mm_scale_body,
        grid=(_N // _B,),
        in_specs=[
            pl.BlockSpec((_B, 1), lambda i: (i, 0)),
            pl.BlockSpec((_B, _D), lambda i: (i, 0)),
            pl.BlockSpec((_D, _D), lambda i: (0, 0)),
        ],
        out_specs=pl.BlockSpec((_B, _D), lambda i: (i, 0)),
        out_shape=jax.ShapeDtypeStruct((_N, _D), jnp.float32),
    )(dinvc, x, w)


def _tc_combine_mm(dinvc, p, g, b, w):
    return pl.pallas_call(
        _combine_mm_body,
        grid=(_N // _B,),
        in_specs=[
            pl.BlockSpec((_B, 1), lambda i: (i, 0)),
            pl.BlockSpec((_NC, _B, _D), lambda i: (0, i, 0)),
            pl.BlockSpec((_B, _D), lambda i: (i, 0)),
            pl.BlockSpec((1, _D), lambda i: (0, 0)),
            pl.BlockSpec((_D, _D), lambda i: (0, 0)),
        ],
        out_specs=pl.BlockSpec((_B, _D), lambda i: (i, 0)),
        out_shape=jax.ShapeDtypeStruct((_N, _D), jnp.float32),
    )(dinvc, p, g, b, w)


def _tc_combine_out(dinvc, p, g, b):
    return pl.pallas_call(
        _combine_out_body,
        grid=(_N // _B,),
        in_specs=[
            pl.BlockSpec((_B, 1), lambda i: (i, 0)),
            pl.BlockSpec((_NC, _B, _D), lambda i: (0, i, 0)),
            pl.BlockSpec((_B, _D), lambda i: (i, 0)),
            pl.BlockSpec((1, _D), lambda i: (0, 0)),
        ],
        out_specs=pl.BlockSpec((_B, _D), lambda i: (i, 0)),
        out_shape=jax.ShapeDtypeStruct((_N, _D), jnp.float32),
    )(dinvc, p, g, b)


def kernel(x, edge_index, W1, b1, W2, b2):
    ei = edge_index.astype(jnp.int32)
    edge3 = ei.reshape(2, _E // _K, _K)
    npad = _TW * _K - (_E - _MAIN * _K)
    it = jnp.arange(npad, dtype=jnp.int32)
    pad2 = jnp.stack([it % _N, _N + it % (_NP - _N)])
    tail3 = jnp.concatenate(
        [ei[:, _MAIN * _K:], pad2], axis=1).reshape(2, _TW, _K)

    ones_k = jnp.ones((_K,), jnp.float32)
    zeros_n = jnp.zeros((_NP,), jnp.float32)
    zeros_row = jnp.zeros((_K, _D), jnp.float32)

    deg2 = _sc_degree(edge3, tail3, ones_k, zeros_n)
    dinvc = _tc_dinv(deg2)

    b1r = b1.reshape(1, _D)
    b2r = b2.reshape(1, _D)

    g1 = _tc_mm_scale(dinvc, x, W1)
    p1 = _sc_aggregate(g1, edge3, tail3, zeros_row)
    g2 = _tc_combine_mm(dinvc, p1, g1, b1r, W2)
    p2 = _sc_aggregate(g2, edge3, tail3, zeros_row)
    return _tc_combine_out(dinvc, p2, g2, b2r)


# speedup vs baseline: 1.3102x; 1.0165x over previous
import functools

import jax
import jax.numpy as jnp
from jax import lax
from jax.experimental import pallas as pl
from jax.experimental.pallas import tpu as pltpu
from jax.experimental.pallas import tpu_sc as plsc

_N = 10000
_E = 320000
_D = 128
_NC = 2
_NS = 16
_NW = _NC * _NS
_K = 128
_EW = 80
_CH = 40
_EPT = _EW * _K
_EP = _NW * _EPT
_NP = 10240
_STRIPE = _NP // _NS


@functools.cache
def _sc_mesh():
    return plsc.VectorSubcoreMesh(core_axis_name="c", subcore_axis_name="s",
                                  num_cores=_NC, num_subcores=_NS)


@functools.cache
def _sc_degree_kernel():
    @functools.partial(
        pl.kernel,
        out_type=jax.ShapeDtypeStruct((_NC, _NP), jnp.float32),
        mesh=_sc_mesh(),
        scratch_types=[
            pltpu.VMEM((_EW, _K), jnp.int32),
            pltpu.VMEM((_K,), jnp.float32),
            pltpu.VMEM_SHARED((_NP,), jnp.float32),
            pltpu.SemaphoreType.DMA,
        ],
    )
    def deg_kernel(edge_hbm, tail_hbm, ones_hbm, zeros_hbm, out_hbm,
                   dst_v, ones_v, deg_sh, sem):
        c = lax.axis_index("c")
        s = lax.axis_index("s")
        wid = c * _NS + s
        pltpu.sync_copy(zeros_hbm.at[pl.ds(s * _STRIPE, _STRIPE)],
                        deg_sh.at[pl.ds(s * _STRIPE, _STRIPE)])
        pltpu.sync_copy(ones_hbm, ones_v)

        @pl.when(wid < _NW - 1)
        def _():
            pltpu.sync_copy(edge_hbm.at[1].at[pl.ds(wid * _EW, _EW)], dst_v)

        @pl.when(wid == _NW - 1)
        def _():
            pltpu.sync_copy(tail_hbm.at[1], dst_v)

        plsc.subcore_barrier()

        @pl.loop(0, _EW)
        def _(j):
            pltpu.async_copy(ones_v, deg_sh.at[dst_v.at[j]], sem, add=True)

        @pl.loop(0, _EW)
        def _(j):
            pltpu.make_async_copy(ones_v, deg_sh.at[dst_v.at[j]], sem).wait()

        plsc.subcore_barrier()
        pltpu.sync_copy(deg_sh.at[pl.ds(s * _STRIPE, _STRIPE)],
                        out_hbm.at[c].at[pl.ds(s * _STRIPE, _STRIPE)])

    return deg_kernel


_MAIN = 2480
_TW = 80


@functools.cache
def _sc_aggregate_kernel():
    @functools.partial(
        pl.kernel,
        out_type=jax.ShapeDtypeStruct((_NC, _NP, _D), jnp.float32),
        mesh=_sc_mesh(),
        scratch_types=[
            pltpu.VMEM((_CH, _K), jnp.int32),
            pltpu.VMEM((_CH, _K), jnp.int32),
            pltpu.VMEM((_K, _D), jnp.float32),
            pltpu.VMEM((_K, _D), jnp.float32),
            pltpu.VMEM_SHARED((_NP, _D), jnp.float32),
            pltpu.SemaphoreType.DMA,
            pltpu.SemaphoreType.DMA,
        ],
    )
    def agg_kernel(g_hbm, edge_hbm, tail_hbm, zrow_hbm, out_hbm,
                   src_v, dst_v, rows_a, rows_b, acc_sh, sem_a, sem_b):
        c = lax.axis_index("c")
        s = lax.axis_index("s")
        base = (c * _NS + s) * _EW
        pltpu.sync_copy(zrow_hbm, rows_a)

        @pl.loop(0, _STRIPE // _K)
        def _(i):
            pltpu.async_copy(rows_a, acc_sh.at[pl.ds(s * _STRIPE + i * _K, _K)],
                             sem_a)

        @pl.loop(0, _STRIPE // _K)
        def _(i):
            pltpu.make_async_copy(
                rows_a, acc_sh.at[pl.ds(s * _STRIPE + i * _K, _K)],
                sem_a).wait()

        plsc.subcore_barrier()

        @pl.loop(0, _EW // _CH)
        def _(ci):
            wstart = base + ci * _CH

            @pl.when(wstart < _MAIN)
            def _():
                pltpu.sync_copy(edge_hbm.at[0].at[pl.ds(wstart, _CH)], src_v)
                pltpu.sync_copy(edge_hbm.at[1].at[pl.ds(wstart, _CH)], dst_v)

            @pl.when(wstart >= _MAIN)
            def _():
                pltpu.sync_copy(
                    tail_hbm.at[0].at[pl.ds(wstart - _MAIN, _CH)], src_v)
                pltpu.sync_copy(
                    tail_hbm.at[1].at[pl.ds(wstart - _MAIN, _CH)], dst_v)

            pltpu.async_copy(g_hbm.at[src_v.at[0]], rows_a, sem_a)
            pltpu.async_copy(g_hbm.at[src_v.at[1]], rows_b, sem_b)

            @pl.loop(0, _CH, step=2)
            def _(j):
                pltpu.make_async_copy(
                    g_hbm.at[src_v.at[j]], rows_a, sem_a).wait()
                pltpu.sync_copy(rows_a, acc_sh.at[dst_v.at[j]], add=True)

                @pl.when(j + 2 < _CH)
                def _():
                    pltpu.async_copy(
                        g_hbm.at[src_v.at[j + 2]], rows_a, sem_a)

                pltpu.make_async_copy(
                    g_hbm.at[src_v.at[j + 1]], rows_b, sem_b).wait()
                pltpu.sync_copy(rows_b, acc_sh.at[dst_v.at[j + 1]],
                                add=True)

                @pl.when(j + 3 < _CH)
                def _():
                    pltpu.async_copy(
                        g_hbm.at[src_v.at[j + 3]], rows_b, sem_b)

        plsc.subcore_barrier()
        pltpu.sync_copy(acc_sh.at[pl.ds(s * _STRIPE, _STRIPE)],
                        out_hbm.at[c].at[pl.ds(s * _STRIPE, _STRIPE)])

    return agg_kernel


def _sc_degree(edge3, tail3, ones_k, zeros_n):
    return _sc_degree_kernel()(edge3, tail3, ones_k, zeros_n)


def _sc_aggregate(g, edge3, tail3, zeros_row):
    return _sc_aggregate_kernel()(g, edge3, tail3, zeros_row)


_B = 2000


def _dinv_body(deg_ref, o_ref):
    ones2 = jnp.ones((2, 1), jnp.float32)
    degsum = lax.dot_general(deg_ref[...], ones2, (((0,), (0,)), ((), ())),
                             preferred_element_type=jnp.float32)
    o_ref[...] = lax.rsqrt(degsum + 1.0)


def _tc_dinv(deg2):
    return pl.pallas_call(
        _dinv_body,
        grid=(1,),
        in_specs=[pl.BlockSpec((2, _NP), lambda i: (0, 0))],
        out_specs=pl.BlockSpec((_NP, 1), lambda i: (0, 0)),
        out_shape=jax.ShapeDtypeStruct((_NP, 1), jnp.float32),
    )(deg2)


def _mm_scale_body(dinv_ref, x_ref, w_ref, o_ref):
    dinv = dinv_ref[...]
    o_ref[...] = jnp.dot(x_ref[...], w_ref[...],
                         preferred_element_type=jnp.float32) * dinv


def _combine_mm_body(dinv_ref, p_ref, g_ref, b_ref, w_ref, o_ref):
    dinv = dinv_ref[...]
    h = jnp.maximum((p_ref[0] + p_ref[1] + g_ref[...]) * dinv + b_ref[...], 0.0)
    o_ref[...] = jnp.dot(h, w_ref[...],
                         preferred_element_type=jnp.float32) * dinv


def _combine_out_body(dinv_ref, p_ref, g_ref, b_ref, o_ref):
    dinv = dinv_ref[...]
    o_ref[...] = jnp.maximum(
        (p_ref[0] + p_ref[1] + g_ref[...]) * dinv + b_ref[...], 0.0)


def _tc_mm_scale(dinvc, x, w):
    return pl.pallas_call(
        _mm_scale_body,
        grid=(_N // _B,),
        in_specs=[
            pl.BlockSpec((_B, 1), lambda i: (i, 0)),
            pl.BlockSpec((_B, _D), lambda i: (i, 0)),
            pl.BlockSpec((_D, _D), lambda i: (0, 0)),
        ],
        out_specs=pl.BlockSpec((_B, _D), lambda i: (i, 0)),
        out_shape=jax.ShapeDtypeStruct((_N, _D), jnp.float32),
    )(dinvc, x, w)


def _tc_combine_mm(dinvc, p, g, b, w):
    return pl.pallas_call(
        _combine_mm_body,
        grid=(_N // _B,),
        in_specs=[
            pl.BlockSpec((_B, 1), lambda i: (i, 0)),
            pl.BlockSpec((_NC, _B, _D), lambda i: (0, i, 0)),
            pl.BlockSpec((_B, _D), lambda i: (i, 0)),
            pl.BlockSpec((1, _D), lambda i: (0, 0)),
            pl.BlockSpec((_D, _D), lambda i: (0, 0)),
        ],
        out_specs=pl.BlockSpec((_B, _D), lambda i: (i, 0)),
        out_shape=jax.ShapeDtypeStruct((_N, _D), jnp.float32),
    )(dinvc, p, g, b, w)


def _tc_combine_out(dinvc, p, g, b):
    return pl.pallas_call(
        _combine_out_body,
        grid=(_N // _B,),
        in_specs=[
            pl.BlockSpec((_B, 1), lambda i: (i, 0)),
            pl.BlockSpec((_NC, _B, _D), lambda i: (0, i, 0)),
            pl.BlockSpec((_B, _D), lambda i: (i, 0)),
            pl.BlockSpec((1, _D), lambda i: (0, 0)),
        ],
        out_specs=pl.BlockSpec((_B, _D), lambda i: (i, 0)),
        out_shape=jax.ShapeDtypeStruct((_N, _D), jnp.float32),
    )(dinvc, p, g, b)


def kernel(x, edge_index, W1, b1, W2, b2):
    ei = edge_index.astype(jnp.int32)
    edge3 = ei.reshape(2, _E // _K, _K)
    npad = _TW * _K - (_E - _MAIN * _K)
    it = jnp.arange(npad, dtype=jnp.int32)
    pad2 = jnp.stack([it % _N, _N + it % (_NP - _N)])
    tail3 = jnp.concatenate(
        [ei[:, _MAIN * _K:], pad2], axis=1).reshape(2, _TW, _K)

    ones_k = jnp.ones((_K,), jnp.float32)
    zeros_n = jnp.zeros((_NP,), jnp.float32)
    zeros_row = jnp.zeros((_K, _D), jnp.float32)

    deg2 = _sc_degree(edge3, tail3, ones_k, zeros_n)
    dinvc = _tc_dinv(deg2)

    b1r = b1.reshape(1, _D)
    b2r = b2.reshape(1, _D)

    g1 = _tc_mm_scale(dinvc, x, W1)
    p1 = _sc_aggregate(g1, edge3, tail3, zeros_row)
    g2 = _tc_combine_mm(dinvc, p1, g1, b1r, W2)
    p2 = _sc_aggregate(g2, edge3, tail3, zeros_row)
    return _tc_combine_out(dinvc, p2, g2, b2r)


# speedup vs baseline: 1.3133x; 1.0024x over previous
import functools

import jax
import jax.numpy as jnp
from jax import lax
from jax.experimental import pallas as pl
from jax.experimental.pallas import tpu as pltpu
from jax.experimental.pallas import tpu_sc as plsc

_N = 10000
_E = 320000
_D = 128
_NC = 2
_NS = 16
_NW = _NC * _NS
_K = 128
_EW = 80
_CH = 40
_NP = 10240
_STRIPE = _NP // _NS


@functools.cache
def _sc_mesh():
    return plsc.VectorSubcoreMesh(core_axis_name="c", subcore_axis_name="s",
                                  num_cores=_NC, num_subcores=_NS)


@functools.cache
def _sc_degree_kernel():
    @functools.partial(
        pl.kernel,
        out_type=jax.ShapeDtypeStruct((_NC, _NP), jnp.float32),
        mesh=_sc_mesh(),
        scratch_types=[
            pltpu.VMEM((_EW, _K), jnp.int32),
            pltpu.VMEM((_K,), jnp.float32),
            pltpu.VMEM_SHARED((_NP,), jnp.float32),
            pltpu.SemaphoreType.DMA,
        ],
    )
    def deg_kernel(edge_hbm, tail_hbm, ones_hbm, zeros_hbm, out_hbm,
                   dst_v, ones_v, deg_sh, sem):
        c = lax.axis_index("c")
        s = lax.axis_index("s")
        wid = c * _NS + s
        pltpu.sync_copy(zeros_hbm.at[pl.ds(s * _STRIPE, _STRIPE)],
                        deg_sh.at[pl.ds(s * _STRIPE, _STRIPE)])
        pltpu.sync_copy(ones_hbm, ones_v)

        @pl.when(wid < _NW - 1)
        def _():
            pltpu.sync_copy(edge_hbm.at[1].at[pl.ds(wid * _EW, _EW)], dst_v)

        @pl.when(wid == _NW - 1)
        def _():
            pltpu.sync_copy(tail_hbm.at[1], dst_v)

        plsc.subcore_barrier()

        @pl.loop(0, _EW)
        def _(j):
            pltpu.async_copy(ones_v, deg_sh.at[dst_v.at[j]], sem, add=True)

        @pl.loop(0, _EW)
        def _(j):
            pltpu.make_async_copy(ones_v, deg_sh.at[dst_v.at[j]], sem).wait()

        plsc.subcore_barrier()
        pltpu.sync_copy(deg_sh.at[pl.ds(s * _STRIPE, _STRIPE)],
                        out_hbm.at[c].at[pl.ds(s * _STRIPE, _STRIPE)])

    return deg_kernel


_MAIN = 2480
_TW = 80


@functools.cache
def _sc_aggregate_kernel():
    @functools.partial(
        pl.kernel,
        out_type=jax.ShapeDtypeStruct((_NC, _NP, _D), jnp.float32),
        mesh=_sc_mesh(),
        scratch_types=[
            pltpu.VMEM((_CH, _K), jnp.int32),
            pltpu.VMEM((_CH, _K), jnp.int32),
            pltpu.VMEM((_K, _D), jnp.float32),
            pltpu.VMEM((_K, _D), jnp.float32),
            pltpu.VMEM_SHARED((_NP, _D), jnp.float32),
            pltpu.SemaphoreType.DMA,
            pltpu.SemaphoreType.DMA,
        ],
    )
    def agg_kernel(g_hbm, edge_hbm, tail_hbm, zrow_hbm, out_hbm,
                   src_v, dst_v, rows_a, rows_b, acc_sh, sem_a, sem_b):
        c = lax.axis_index("c")
        s = lax.axis_index("s")
        base = (c * _NS + s) * _EW
        pltpu.sync_copy(zrow_hbm, rows_a)

        @pl.loop(0, _STRIPE // _K)
        def _(i):
            pltpu.async_copy(rows_a, acc_sh.at[pl.ds(s * _STRIPE + i * _K, _K)],
                             sem_a)

        @pl.loop(0, _STRIPE // _K)
        def _(i):
            pltpu.make_async_copy(
                rows_a, acc_sh.at[pl.ds(s * _STRIPE + i * _K, _K)],
                sem_a).wait()

        plsc.subcore_barrier()

        @pl.loop(0, _EW // _CH)
        def _(ci):
            wstart = base + ci * _CH

            @pl.when(wstart < _MAIN)
            def _():
                pltpu.sync_copy(edge_hbm.at[0].at[pl.ds(wstart, _CH)], src_v)
                pltpu.sync_copy(edge_hbm.at[1].at[pl.ds(wstart, _CH)], dst_v)

            @pl.when(wstart >= _MAIN)
            def _():
                pltpu.sync_copy(
                    tail_hbm.at[0].at[pl.ds(wstart - _MAIN, _CH)], src_v)
                pltpu.sync_copy(
                    tail_hbm.at[1].at[pl.ds(wstart - _MAIN, _CH)], dst_v)

            pltpu.async_copy(g_hbm.at[src_v.at[0]], rows_a, sem_a)
            pltpu.async_copy(g_hbm.at[src_v.at[1]], rows_b, sem_b)

            @pl.loop(0, _CH, step=2)
            def _(j):
                pltpu.make_async_copy(
                    g_hbm.at[src_v.at[j]], rows_a, sem_a).wait()
                pltpu.sync_copy(rows_a, acc_sh.at[dst_v.at[j]], add=True)

                @pl.when(j + 2 < _CH)
                def _():
                    pltpu.async_copy(
                        g_hbm.at[src_v.at[j + 2]], rows_a, sem_a)

                pltpu.make_async_copy(
                    g_hbm.at[src_v.at[j + 1]], rows_b, sem_b).wait()
                pltpu.sync_copy(rows_b, acc_sh.at[dst_v.at[j + 1]],
                                add=True)

                @pl.when(j + 3 < _CH)
                def _():
                    pltpu.async_copy(
                        g_hbm.at[src_v.at[j + 3]], rows_b, sem_b)

        plsc.subcore_barrier()
        pltpu.sync_copy(acc_sh.at[pl.ds(s * _STRIPE, _STRIPE)],
                        out_hbm.at[c].at[pl.ds(s * _STRIPE, _STRIPE)])

    return agg_kernel


def _sc_degree(edge3, tail3, ones_k, zeros_n):
    return _sc_degree_kernel()(edge3, tail3, ones_k, zeros_n)


def _sc_aggregate(g, edge3, tail3, zeros_row):
    return _sc_aggregate_kernel()(g, edge3, tail3, zeros_row)


_B = 2000


def _dinv_body(deg_ref, o_ref):
    ones2 = jnp.ones((2, 1), jnp.float32)
    degsum = lax.dot_general(deg_ref[...], ones2, (((0,), (0,)), ((), ())),
                             preferred_element_type=jnp.float32)
    o_ref[...] = lax.rsqrt(degsum + 1.0)


def _tc_dinv(deg2):
    return pl.pallas_call(
        _dinv_body,
        grid=(1,),
        in_specs=[pl.BlockSpec((2, _NP), lambda i: (0, 0))],
        out_specs=pl.BlockSpec((_NP, 1), lambda i: (0, 0)),
        out_shape=jax.ShapeDtypeStruct((_NP, 1), jnp.float32),
    )(deg2)


def _mm_scale_body(dinv_ref, x_ref, w_ref, o_ref):
    dinv = dinv_ref[...]
    o_ref[...] = jnp.dot(x_ref[...], w_ref[...],
                         preferred_element_type=jnp.float32) * dinv


def _combine_mm_body(dinv_ref, p_ref, g_ref, b_ref, w_ref, o_ref):
    dinv = dinv_ref[...]
    h = jnp.maximum((p_ref[0] + p_ref[1] + g_ref[...]) * dinv + b_ref[...], 0.0)
    o_ref[...] = jnp.dot(h, w_ref[...],
                         preferred_element_type=jnp.float32) * dinv


def _combine_out_body(dinv_ref, p_ref, g_ref, b_ref, o_ref):
    dinv = dinv_ref[...]
    o_ref[...] = jnp.maximum(
        (p_ref[0] + p_ref[1] + g_ref[...]) * dinv + b_ref[...], 0.0)


def _tc_mm_scale(dinvc, x, w):
    return pl.pallas_call(
        _mm_scale_body,
        grid=(_N // _B,),
        in_specs=[
            pl.BlockSpec((_B, 1), lambda i: (i, 0)),
            pl.BlockSpec((_B, _D), lambda i: (i, 0)),
            pl.BlockSpec((_D, _D), lambda i: (0, 0)),
        ],
        out_specs=pl.BlockSpec((_B, _D), lambda i: (i, 0)),
        out_shape=jax.ShapeDtypeStruct((_N, _D), jnp.float32),
    )(dinvc, x, w)


def _tc_combine_mm(dinvc, p, g, b, w):
    return pl.pallas_call(
        _combine_mm_body,
        grid=(_N // _B,),
        in_specs=[
            pl.BlockSpec((_B, 1), lambda i: (i, 0)),
            pl.BlockSpec((_NC, _B, _D), lambda i: (0, i, 0)),
            pl.BlockSpec((_B, _D), lambda i: (i, 0)),
            pl.BlockSpec((1, _D), lambda i: (0, 0)),
            pl.BlockSpec((_D, _D), lambda i: (0, 0)),
        ],
        out_specs=pl.BlockSpec((_B, _D), lambda i: (i, 0)),
        out_shape=jax.ShapeDtypeStruct((_N, _D), jnp.float32),
    )(dinvc, p, g, b, w)


def _tc_combine_out(dinvc, p, g, b):
    return pl.pallas_call(
        _combine_out_body,
        grid=(_N // _B,),
        in_specs=[
            pl.BlockSpec((_B, 1), lambda i: (i, 0)),
            pl.BlockSpec((_NC, _B, _D), lambda i: (0, i, 0)),
            pl.BlockSpec((_B, _D), lambda i: (i, 0)),
            pl.BlockSpec((1, _D), lambda i: (0, 0)),
        ],
        out_specs=pl.BlockSpec((_B, _D), lambda i: (i, 0)),
        out_shape=jax.ShapeDtypeStruct((_N, _D), jnp.float32),
    )(dinvc, p, g, b)


def kernel(x, edge_index, W1, b1, W2, b2):
    ei = edge_index.astype(jnp.int32)
    edge3 = ei.reshape(2, _E // _K, _K)
    npad = _TW * _K - (_E - _MAIN * _K)
    it = jnp.arange(npad, dtype=jnp.int32)
    pad2 = jnp.stack([it % _N, _N + it % (_NP - _N)])
    tail3 = jnp.concatenate(
        [ei[:, _MAIN * _K:], pad2], axis=1).reshape(2, _TW, _K)

    ones_k = jnp.ones((_K,), jnp.float32)
    zeros_n = jnp.zeros((_NP,), jnp.float32)
    zeros_row = jnp.zeros((_K, _D), jnp.float32)

    deg2 = _sc_degree(edge3, tail3, ones_k, zeros_n)
    dinvc = _tc_dinv(deg2)

    b1r = b1.reshape(1, _D)
    b2r = b2.reshape(1, _D)

    g1 = _tc_mm_scale(dinvc, x, W1)
    p1 = _sc_aggregate(g1, edge3, tail3, zeros_row)
    g2 = _tc_combine_mm(dinvc, p1, g1, b1r, W2)
    p2 = _sc_aggregate(g2, edge3, tail3, zeros_row)
    return _tc_combine_out(dinvc, p2, g2, b2r)
